# baseline (speedup 1.0000x reference)
import functools

import jax
import jax.numpy as jnp
from jax import lax
from jax.experimental import pallas as pl
from jax.experimental.pallas import tpu as pltpu
from jax.experimental.pallas import tpu_sc as plsc

NUM_CORES = 2
NUM_SUBCORES = 16
NUM_WORKERS = NUM_CORES * NUM_SUBCORES
LANES = 16

BATCH = 16384
EMBED_DIM = 64
CHUNKS = EMBED_DIM // LANES
BPW = BATCH // NUM_WORKERS
GROUPS = BPW // LANES


def _mf_body(iu_hbm, ii_hbm, ue_hbm, ie_hbm, ub_hbm, ib_hbm, out_hbm,
             idx_u_v, idx_i_v, u_rows, i_rows, bu_v, bi_v, out_v, sem):
    c = lax.axis_index("c")
    s = lax.axis_index("s")
    wid = s * NUM_CORES + c
    base = wid * BPW

    pltpu.sync_copy(iu_hbm.at[pl.ds(base, BPW)], idx_u_v)
    pltpu.sync_copy(ii_hbm.at[pl.ds(base, BPW)], idx_i_v)

    cp_u = pltpu.async_copy(ue_hbm.at[idx_u_v], u_rows, sem)
    cp_i = pltpu.async_copy(ie_hbm.at[idx_i_v], i_rows, sem)
    cp_bu = pltpu.async_copy(ub_hbm.at[idx_u_v], bu_v, sem)
    cp_bi = pltpu.async_copy(ib_hbm.at[idx_i_v], bi_v, sem)
    cp_u.wait()
    cp_i.wait()
    cp_bu.wait()
    cp_bi.wait()

    iota = lax.iota(jnp.int32, LANES)
    rot_idx = [(iota + d) % LANES for d in (8, 4, 2, 1)]

    def group(g, carry):
        res = jnp.zeros((LANES,), jnp.float32)
        for j in range(LANES):
            r = g * LANES + j
            acc = u_rows[r, pl.ds(0, LANES)] * i_rows[r, pl.ds(0, LANES)]
            for k in range(1, CHUNKS):
                acc = acc + (u_rows[r, pl.ds(k * LANES, LANES)]
                             * i_rows[r, pl.ds(k * LANES, LANES)])
            for ridx in rot_idx:
                acc = acc + acc[ridx]
            res = jnp.where(iota == j, acc, res)
        sl = pl.ds(g * LANES, LANES)
        out_v[sl] = res + bu_v[sl] + bi_v[sl]
        return carry
    lax.fori_loop(0, GROUPS, group, 0)

    pltpu.sync_copy(out_v, out_hbm.at[pl.ds(base, BPW)])


@jax.jit
def _mf_call(idx_user, idx_item, user_emb, item_emb, user_bias, item_bias):
    mesh = plsc.VectorSubcoreMesh(
        core_axis_name="c", subcore_axis_name="s",
        num_cores=NUM_CORES, num_subcores=NUM_SUBCORES)
    fn = pl.kernel(
        _mf_body,
        out_type=jax.ShapeDtypeStruct((BATCH,), jnp.float32),
        mesh=mesh,
        scratch_types=[
            pltpu.VMEM((BPW,), jnp.int32),
            pltpu.VMEM((BPW,), jnp.int32),
            pltpu.VMEM((BPW, EMBED_DIM), jnp.float32),
            pltpu.VMEM((BPW, EMBED_DIM), jnp.float32),
            pltpu.VMEM((BPW,), jnp.float32),
            pltpu.VMEM((BPW,), jnp.float32),
            pltpu.VMEM((BPW,), jnp.float32),
            pltpu.SemaphoreType.DMA,
        ],
        compiler_params=pltpu.CompilerParams(use_tc_tiling_on_sc=False),
    )
    return fn(idx_user, idx_item, user_emb, item_emb, user_bias, item_bias)


def kernel(x, user_emb, item_emb, user_bias, item_bias):
    xi = x.astype(jnp.int32)
    return _mf_call(xi[:, 0], xi[:, 1], user_emb, item_emb,
                    user_bias.reshape(-1), item_bias.reshape(-1))
